# Initial kernel scaffold; baseline (speedup 1.0000x reference)
import jax
import jax.numpy as jnp
from jax.experimental import pallas as pl


def kernel(x, edge_index, W, b):
    raise NotImplementedError("write your pallas kernel here")


# speedup vs baseline: 12.7560x; 12.7560x over previous
import functools

import jax
import jax.numpy as jnp
from jax import lax
from jax.experimental import pallas as pl
from jax.experimental.pallas import tpu as pltpu
from jax.experimental.pallas import tpu_sc as plsc

N = 10000
E = 320000
D = 128
DH = D // 2
NC, NS, L = 2, 16, 16
NPAD = 10240
PT = NPAD // NS
EPT1 = E // NS
C1 = 4000
EPT3 = E // (NC * NS)
C3 = 80
NPR = N // NS

_MESH = plsc.VectorSubcoreMesh(
    core_axis_name="c", subcore_axis_name="s", num_cores=NC, num_subcores=NS)
_SC_PARAMS = pltpu.CompilerParams(needs_layout_passes=False)


def _zero_1d(ref, nwords):
    z = jnp.zeros((L,), jnp.float32)

    def body(i, _):
        ref[pl.ds(i * L, L)] = z
        return 0

    lax.fori_loop(0, nwords // L, body, 0)


@functools.partial(
    pl.kernel,
    out_type=jax.ShapeDtypeStruct((2, NPAD), jnp.float32),
    mesh=_MESH,
    compiler_params=_SC_PARAMS,
    scratch_types=[
        pltpu.VMEM_SHARED((NS, NPAD), jnp.float32),
        pltpu.VMEM((NPAD,), jnp.float32),
        pltpu.VMEM((C1,), jnp.int32),
        pltpu.VMEM((PT,), jnp.float32),
        pltpu.VMEM((PT,), jnp.float32),
    ],
)
def _norms_kernel(ei, norms, part_sh, loc, idxb, tmp, accb):
    c = lax.axis_index("c")
    s = lax.axis_index("s")
    _zero_1d(loc, NPAD)
    ones = jnp.ones((L,), jnp.float32)
    eoff = s * EPT1

    def chunk_body(q, _):
        pltpu.sync_copy(ei.at[pl.ds(c * E + eoff + q * C1, C1)], idxb)

        def inner(k, _):
            v = idxb[pl.ds(k * L, L)]
            plsc.addupdate_scatter(loc, [v], ones)
            return 0

        lax.fori_loop(0, C1 // L, inner, 0)
        return 0

    lax.fori_loop(0, EPT1 // C1, chunk_body, 0)
    pltpu.sync_copy(loc, part_sh.at[s])
    plsc.subcore_barrier()

    noff = s * PT
    pltpu.sync_copy(part_sh.at[0, pl.ds(noff, PT)], accb)

    def red(p, _):
        pltpu.sync_copy(part_sh.at[p, pl.ds(noff, PT)], tmp)

        def add_(j, _):
            sl = pl.ds(j * L, L)
            accb[sl] = accb[sl] + tmp[sl]
            return 0

        lax.fori_loop(0, PT // L, add_, 0)
        return 0

    lax.fori_loop(1, NS, red, 0)

    def nrm(j, _):
        sl = pl.ds(j * L, L)
        d = jnp.maximum(accb[sl], 1.0)
        i = plsc.bitcast(d, jnp.int32)
        i = 0x5F3759DF - lax.shift_right_logical(i, 1)
        y = plsc.bitcast(i, jnp.float32)
        for _ in range(3):
            y = y * (1.5 - 0.5 * d * y * y)
        accb[sl] = y
        return 0

    lax.fori_loop(0, PT // L, nrm, 0)
    pltpu.sync_copy(accb, norms.at[c, pl.ds(noff, PT)])


@functools.partial(
    pl.kernel,
    out_type=jax.ShapeDtypeStruct((2, NPAD, D), jnp.float32),
    mesh=_MESH,
    compiler_params=_SC_PARAMS,
    scratch_types=[
        pltpu.VMEM_SHARED((NPAD, D), jnp.float32),
        pltpu.VMEM((C3,), jnp.int32),
        pltpu.VMEM((C3,), jnp.int32),
        pltpu.VMEM((C3, D), jnp.float32),
        pltpu.SemaphoreType.DMA,
    ],
)
def _agg_kernel(hs, ei, out, acc_sh, sidx, didx, rows, sem):
    c = lax.axis_index("c")
    s = lax.axis_index("s")

    z = jnp.zeros((L,), jnp.float32)

    def zr(r, _):
        def zc(j, _):
            rows[r, pl.ds(j * L, L)] = z
            return 0

        lax.fori_loop(0, D // L, zc, 0)
        return 0

    lax.fori_loop(0, C3, zr, 0)

    def za(m, _):
        pltpu.sync_copy(rows, acc_sh.at[pl.ds(s * PT + m * C3, C3)])
        return 0

    lax.fori_loop(0, PT // C3, za, 0)
    plsc.subcore_barrier()

    eoff = (c * NS + s) * EPT3

    def chunk(q, _):
        base = eoff + q * C3
        pltpu.sync_copy(ei.at[pl.ds(base, C3)], sidx)
        pltpu.sync_copy(ei.at[pl.ds(E + base, C3)], didx)
        pltpu.async_copy(hs.at[sidx], rows, sem).wait()
        pltpu.sync_copy(rows, acc_sh.at[didx], add=True)
        return 0

    lax.fori_loop(0, EPT3 // C3, chunk, 0)
    plsc.subcore_barrier()
    pltpu.sync_copy(acc_sh.at[pl.ds(s * PT, PT)],
                    out.at[c, pl.ds(s * PT, PT)])


_BLK = 1000


def _scale_matmul_body(x_ref, ns_ref, w_ref, o_ref):
    o_ref[...] = jnp.dot(x_ref[...] * ns_ref[...], w_ref[...],
                         preferred_element_type=jnp.float32)


def _final_body(a0_ref, a1_ref, nd_ref, b_ref, o_ref):
    o_ref[...] = (a0_ref[0] + a1_ref[0]) * nd_ref[...] + b_ref[...]


def kernel(x, edge_index, W, b):
    ei_flat = edge_index.reshape(2 * E)
    norms = _norms_kernel(ei_flat)
    ns = norms[0, :N][:, None]
    nd = norms[1, :N][:, None]

    hs = pl.pallas_call(
        _scale_matmul_body,
        grid=(N // _BLK,),
        in_specs=[
            pl.BlockSpec((_BLK, D), lambda i: (i, 0)),
            pl.BlockSpec((_BLK, 1), lambda i: (i, 0)),
            pl.BlockSpec((D, D), lambda i: (0, 0)),
        ],
        out_specs=pl.BlockSpec((_BLK, D), lambda i: (i, 0)),
        out_shape=jax.ShapeDtypeStruct((N, D), jnp.float32),
    )(x, ns, W)

    agg2 = _agg_kernel(hs, ei_flat)[:, :N, :]

    out = pl.pallas_call(
        _final_body,
        grid=(N // _BLK,),
        in_specs=[
            pl.BlockSpec((1, _BLK, D), lambda i: (0, i, 0)),
            pl.BlockSpec((1, _BLK, D), lambda i: (1, i, 0)),
            pl.BlockSpec((_BLK, 1), lambda i: (i, 0)),
            pl.BlockSpec((1, D), lambda i: (0, 0)),
        ],
        out_specs=pl.BlockSpec((_BLK, D), lambda i: (i, 0)),
        out_shape=jax.ShapeDtypeStruct((N, D), jnp.float32),
    )(agg2, agg2, nd, b.reshape(1, D))
    return out


# speedup vs baseline: 19.0876x; 1.4964x over previous
import functools

import jax
import jax.numpy as jnp
from jax import lax
from jax.experimental import pallas as pl
from jax.experimental.pallas import tpu as pltpu
from jax.experimental.pallas import tpu_sc as plsc

N = 10000
E = 320000
D = 128
DH = D // 2
NC, NS, L = 2, 16, 16
NPAD = 10240
PT = NPAD // NS
EPT1 = E // NS
C1 = 4000
EPT3 = E // (NC * NS)
C3 = 80
NPR = N // NS

_MESH = plsc.VectorSubcoreMesh(
    core_axis_name="c", subcore_axis_name="s", num_cores=NC, num_subcores=NS)
_SC_PARAMS = pltpu.CompilerParams(needs_layout_passes=False)


def _zero_1d(ref, nwords):
    z = jnp.zeros((L,), jnp.float32)

    def body(i, _):
        ref[pl.ds(i * L, L)] = z
        return 0

    lax.fori_loop(0, nwords // L, body, 0)


@functools.partial(
    pl.kernel,
    out_type=jax.ShapeDtypeStruct((2, NPAD), jnp.float32),
    mesh=_MESH,
    compiler_params=_SC_PARAMS,
    scratch_types=[
        pltpu.VMEM_SHARED((NS, NPAD), jnp.float32),
        pltpu.VMEM((NPAD,), jnp.float32),
        pltpu.VMEM((C1,), jnp.int32),
        pltpu.VMEM((PT,), jnp.float32),
        pltpu.VMEM((PT,), jnp.float32),
    ],
)
def _norms_kernel(ei, norms, part_sh, loc, idxb, tmp, accb):
    c = lax.axis_index("c")
    s = lax.axis_index("s")
    _zero_1d(loc, NPAD)
    ones = jnp.ones((L,), jnp.float32)
    eoff = s * EPT1

    def chunk_body(q, _):
        pltpu.sync_copy(ei.at[pl.ds(c * E + eoff + q * C1, C1)], idxb)

        def inner(k, _):
            v = idxb[pl.ds(k * L, L)]
            plsc.addupdate_scatter(loc, [v], ones)
            return 0

        lax.fori_loop(0, C1 // L, inner, 0)
        return 0

    lax.fori_loop(0, EPT1 // C1, chunk_body, 0)
    pltpu.sync_copy(loc, part_sh.at[s])
    plsc.subcore_barrier()

    noff = s * PT
    pltpu.sync_copy(part_sh.at[0, pl.ds(noff, PT)], accb)

    def red(p, _):
        pltpu.sync_copy(part_sh.at[p, pl.ds(noff, PT)], tmp)

        def add_(j, _):
            sl = pl.ds(j * L, L)
            accb[sl] = accb[sl] + tmp[sl]
            return 0

        lax.fori_loop(0, PT // L, add_, 0)
        return 0

    lax.fori_loop(1, NS, red, 0)

    def nrm(j, _):
        sl = pl.ds(j * L, L)
        d = jnp.maximum(accb[sl], 1.0)
        i = plsc.bitcast(d, jnp.int32)
        i = 0x5F3759DF - lax.shift_right_logical(i, 1)
        y = plsc.bitcast(i, jnp.float32)
        for _ in range(3):
            y = y * (1.5 - 0.5 * d * y * y)
        accb[sl] = y
        return 0

    lax.fori_loop(0, PT // L, nrm, 0)
    pltpu.sync_copy(accb, norms.at[c, pl.ds(noff, PT)])


NCH = EPT3 // C3


@functools.partial(
    pl.kernel,
    out_type=jax.ShapeDtypeStruct((2, NPAD, D), jnp.float32),
    mesh=_MESH,
    compiler_params=_SC_PARAMS,
    scratch_types=[
        pltpu.VMEM_SHARED((NPAD, D), jnp.float32),
        pltpu.VMEM((C3,), jnp.int32),
        pltpu.VMEM((C3,), jnp.int32),
        pltpu.VMEM((C3,), jnp.int32),
        pltpu.VMEM((C3,), jnp.int32),
        pltpu.VMEM((C3, D), jnp.float32),
        pltpu.VMEM((C3, D), jnp.float32),
        pltpu.SemaphoreType.DMA,
        pltpu.SemaphoreType.DMA,
        pltpu.SemaphoreType.DMA,
        pltpu.SemaphoreType.DMA,
    ],
)
def _agg_kernel(hs, ei, out, acc_sh, sidx0, sidx1, didx0, didx1,
                rows0, rows1, isem0, isem1, gsem0, gsem1):
    c = lax.axis_index("c")
    s = lax.axis_index("s")
    eoff = (c * NS + s) * EPT3

    z = jnp.zeros((L,), jnp.float32)

    def zr(r, _):
        def zc(j, _):
            rows0[r, pl.ds(j * L, L)] = z
            return 0

        lax.fori_loop(0, D // L, zc, 0)
        return 0

    lax.fori_loop(0, C3, zr, 0)

    def za(m, _):
        pltpu.sync_copy(rows0, acc_sh.at[pl.ds(s * PT + m * C3, C3)])
        return 0

    lax.fori_loop(0, PT // C3, za, 0)
    plsc.subcore_barrier()

    def idx_load(q, sb, db):
        base = eoff + q * C3
        pltpu.sync_copy(ei.at[pl.ds(base, C3)], sb)
        pltpu.sync_copy(ei.at[pl.ds(E + base, C3)], db)

    def gather_start(sb, buf, gsem):
        pltpu.async_copy(hs.at[sb], buf, gsem)

    def gather_wait(sb, buf, gsem):
        pltpu.make_async_copy(hs.at[sb], buf, gsem).wait()

    def scatter(db, buf):
        pltpu.sync_copy(buf, acc_sh.at[db], add=True)

    idx_load(0, sidx0, didx0)
    gather_start(sidx0, rows0, gsem0)

    def steady(i, _):
        idx_load(2 * i + 1, sidx1, didx1)
        gather_start(sidx1, rows1, gsem1)
        gather_wait(sidx0, rows0, gsem0)
        scatter(didx0, rows0)
        idx_load(2 * i + 2, sidx0, didx0)
        gather_start(sidx0, rows0, gsem0)
        gather_wait(sidx1, rows1, gsem1)
        scatter(didx1, rows1)
        return 0

    lax.fori_loop(0, (NCH - 1) // 2, steady, 0)
    gather_wait(sidx0, rows0, gsem0)
    scatter(didx0, rows0)

    plsc.subcore_barrier()
    pltpu.sync_copy(acc_sh.at[pl.ds(s * PT, PT)],
                    out.at[c, pl.ds(s * PT, PT)])


_BLK = 1000


def _scale_matmul_body(x_ref, ns_ref, w_ref, o_ref):
    o_ref[...] = jnp.dot(x_ref[...] * ns_ref[...], w_ref[...],
                         preferred_element_type=jnp.float32)


def _final_body(a0_ref, a1_ref, nd_ref, b_ref, o_ref):
    o_ref[...] = (a0_ref[0] + a1_ref[0]) * nd_ref[...] + b_ref[...]


def kernel(x, edge_index, W, b):
    ei_flat = edge_index.reshape(2 * E)
    norms = _norms_kernel(ei_flat)
    ns = norms[0, :N][:, None]
    nd = norms[1, :N][:, None]

    hs = pl.pallas_call(
        _scale_matmul_body,
        grid=(N // _BLK,),
        in_specs=[
            pl.BlockSpec((_BLK, D), lambda i: (i, 0)),
            pl.BlockSpec((_BLK, 1), lambda i: (i, 0)),
            pl.BlockSpec((D, D), lambda i: (0, 0)),
        ],
        out_specs=pl.BlockSpec((_BLK, D), lambda i: (i, 0)),
        out_shape=jax.ShapeDtypeStruct((N, D), jnp.float32),
    )(x, ns, W)

    agg2 = _agg_kernel(hs, ei_flat)[:, :N, :]

    out = pl.pallas_call(
        _final_body,
        grid=(N // _BLK,),
        in_specs=[
            pl.BlockSpec((1, _BLK, D), lambda i: (0, i, 0)),
            pl.BlockSpec((1, _BLK, D), lambda i: (1, i, 0)),
            pl.BlockSpec((_BLK, 1), lambda i: (i, 0)),
            pl.BlockSpec((1, D), lambda i: (0, 0)),
        ],
        out_specs=pl.BlockSpec((_BLK, D), lambda i: (i, 0)),
        out_shape=jax.ShapeDtypeStruct((N, D), jnp.float32),
    )(agg2, agg2, nd, b.reshape(1, D))
    return out


# speedup vs baseline: 23.7943x; 1.2466x over previous
import functools

import jax
import jax.numpy as jnp
from jax import lax
from jax.experimental import pallas as pl
from jax.experimental.pallas import tpu as pltpu
from jax.experimental.pallas import tpu_sc as plsc

N = 10000
E = 320000
D = 128
DH = D // 2
NC, NS, L = 2, 16, 16
NPAD = 10240
PT = NPAD // NS
EPT1 = E // NS
C1 = 2000
EPT3 = E // (NC * NS)
C3 = 80
NPR = N // NS

_MESH = plsc.VectorSubcoreMesh(
    core_axis_name="c", subcore_axis_name="s", num_cores=NC, num_subcores=NS)
_SC_PARAMS = pltpu.CompilerParams(needs_layout_passes=False)


def _zero_1d(ref, nwords):
    z = jnp.zeros((L,), jnp.float32)

    def body(i, _):
        ref[pl.ds(i * L, L)] = z
        return 0

    lax.fori_loop(0, nwords // L, body, 0)


@functools.partial(
    pl.kernel,
    out_type=jax.ShapeDtypeStruct((2, NPAD), jnp.float32),
    mesh=_MESH,
    compiler_params=_SC_PARAMS,
    scratch_types=[
        pltpu.VMEM_SHARED((NS, NPAD), jnp.float32),
        pltpu.VMEM((NPAD,), jnp.float32),
        pltpu.VMEM((C1,), jnp.int32),
        pltpu.VMEM((PT,), jnp.float32),
        pltpu.VMEM((PT,), jnp.float32),
    ],
)
def _norms_kernel(ei, norms, part_sh, loc, idxb, tmp, accb):
    c = lax.axis_index("c")
    s = lax.axis_index("s")
    _zero_1d(loc, NPAD)
    ones = jnp.ones((L,), jnp.float32)
    eoff = s * EPT1

    def chunk_body(q, _):
        pltpu.sync_copy(ei.at[pl.ds(c * E + eoff + q * C1, C1)], idxb)

        def inner(k, _):
            v = idxb[pl.ds(k * L, L)]
            plsc.addupdate_scatter(loc, [v], ones)
            return 0

        lax.fori_loop(0, C1 // L, inner, 0)
        return 0

    lax.fori_loop(0, EPT1 // C1, chunk_body, 0)
    pltpu.sync_copy(loc, part_sh.at[s])
    plsc.subcore_barrier()

    noff = s * PT
    pltpu.sync_copy(part_sh.at[0, pl.ds(noff, PT)], accb)

    def red(p, _):
        pltpu.sync_copy(part_sh.at[p, pl.ds(noff, PT)], tmp)

        def add_(j, _):
            sl = pl.ds(j * L, L)
            accb[sl] = accb[sl] + tmp[sl]
            return 0

        lax.fori_loop(0, PT // L, add_, 0)
        return 0

    lax.fori_loop(1, NS, red, 0)

    def nrm(j, _):
        sl = pl.ds(j * L, L)
        d = jnp.maximum(accb[sl], 1.0)
        i = plsc.bitcast(d, jnp.int32)
        i = 0x5F3759DF - lax.shift_right_logical(i, 1)
        y = plsc.bitcast(i, jnp.float32)
        for _ in range(3):
            y = y * (1.5 - 0.5 * d * y * y)
        accb[sl] = y
        return 0

    lax.fori_loop(0, PT // L, nrm, 0)
    pltpu.sync_copy(accb, norms.at[c, pl.ds(noff, PT)])


NCH = EPT3 // C3
BC = 25
NBT = NCH // BC


@functools.partial(
    pl.kernel,
    out_type=jax.ShapeDtypeStruct((2, NPAD, D), jnp.float32),
    mesh=_MESH,
    compiler_params=_SC_PARAMS,
    scratch_types=[
        pltpu.VMEM_SHARED((NPAD, D), jnp.float32),
        pltpu.VMEM((BC, C3), jnp.int32),
        pltpu.VMEM((BC, C3), jnp.int32),
        pltpu.VMEM((C3, D), jnp.float32),
        pltpu.VMEM((C3, D), jnp.float32),
        pltpu.SemaphoreType.DMA,
        pltpu.SemaphoreType.DMA,
    ],
)
def _agg_kernel(hs, src3, dst3, out, acc_sh, sidxb, didxb,
                rows0, rows1, gsem0, gsem1):
    c = lax.axis_index("c")
    s = lax.axis_index("s")
    w = c * NS + s

    z = jnp.zeros((L,), jnp.float32)

    def zr(r, _):
        def zc(j, _):
            rows0[r, pl.ds(j * L, L)] = z
            return 0

        lax.fori_loop(0, D // L, zc, 0)
        return 0

    lax.fori_loop(0, C3, zr, 0)

    def za(m, _):
        pltpu.sync_copy(rows0, acc_sh.at[pl.ds(s * PT + m * C3, C3)])
        return 0

    lax.fori_loop(0, PT // C3, za, 0)
    plsc.subcore_barrier()

    def gather_start(q, buf, gsem):
        pltpu.async_copy(hs.at[sidxb.at[q]], buf, gsem)

    def gather_wait(q, buf, gsem):
        pltpu.make_async_copy(hs.at[sidxb.at[q]], buf, gsem).wait()

    def scatter(q, buf):
        pltpu.sync_copy(buf, acc_sh.at[didxb.at[q]], add=True)

    def block(b, _):
        bb = w * NBT + b
        pltpu.sync_copy(src3.at[bb], sidxb)
        pltpu.sync_copy(dst3.at[bb], didxb)
        gather_start(0, rows0, gsem0)

        def pair(i, _):
            gather_start(2 * i + 1, rows1, gsem1)
            gather_wait(2 * i, rows0, gsem0)
            scatter(2 * i, rows0)
            gather_start(2 * i + 2, rows0, gsem0)
            gather_wait(2 * i + 1, rows1, gsem1)
            scatter(2 * i + 1, rows1)
            return 0

        lax.fori_loop(0, (BC - 1) // 2, pair, 0)
        gather_wait(BC - 1, rows0, gsem0)
        scatter(BC - 1, rows0)
        return 0

    lax.fori_loop(0, NBT, block, 0)

    plsc.subcore_barrier()
    pltpu.sync_copy(acc_sh.at[pl.ds(s * PT, PT)],
                    out.at[c, pl.ds(s * PT, PT)])


_BLK = 1000


def _scale_matmul_body(x_ref, ns_ref, w_ref, o_ref):
    o_ref[...] = jnp.dot(x_ref[...] * ns_ref[...], w_ref[...],
                         preferred_element_type=jnp.float32)


def _final_body(a0_ref, a1_ref, nd_ref, b_ref, o_ref):
    o_ref[...] = (a0_ref[0] + a1_ref[0]) * nd_ref[...] + b_ref[...]


def kernel(x, edge_index, W, b):
    ei_flat = edge_index.reshape(2 * E)
    norms = _norms_kernel(ei_flat)
    ns = norms[0, :N][:, None]
    nd = norms[1, :N][:, None]

    hs = pl.pallas_call(
        _scale_matmul_body,
        grid=(N // _BLK,),
        in_specs=[
            pl.BlockSpec((_BLK, D), lambda i: (i, 0)),
            pl.BlockSpec((_BLK, 1), lambda i: (i, 0)),
            pl.BlockSpec((D, D), lambda i: (0, 0)),
        ],
        out_specs=pl.BlockSpec((_BLK, D), lambda i: (i, 0)),
        out_shape=jax.ShapeDtypeStruct((N, D), jnp.float32),
    )(x, ns, W)

    src3 = edge_index[0].reshape(NC * NS * NBT, BC, C3)
    dst3 = edge_index[1].reshape(NC * NS * NBT, BC, C3)
    agg2 = _agg_kernel(hs, src3, dst3)[:, :N, :]

    out = pl.pallas_call(
        _final_body,
        grid=(N // _BLK,),
        in_specs=[
            pl.BlockSpec((1, _BLK, D), lambda i: (0, i, 0)),
            pl.BlockSpec((1, _BLK, D), lambda i: (1, i, 0)),
            pl.BlockSpec((_BLK, 1), lambda i: (i, 0)),
            pl.BlockSpec((1, D), lambda i: (0, 0)),
        ],
        out_specs=pl.BlockSpec((_BLK, D), lambda i: (i, 0)),
        out_shape=jax.ShapeDtypeStruct((N, D), jnp.float32),
    )(agg2, agg2, nd, b.reshape(1, D))
    return out


# speedup vs baseline: 25.1899x; 1.0587x over previous
import functools

import jax
import jax.numpy as jnp
from jax import lax
from jax.experimental import pallas as pl
from jax.experimental.pallas import tpu as pltpu
from jax.experimental.pallas import tpu_sc as plsc

N = 10000
E = 320000
D = 128
DH = D // 2
NC, NS, L = 2, 16, 16
NPAD = 10240
PT = NPAD // NS
EPT1 = E // NS
C1 = 2000
EPT3 = E // (NC * NS)
C3 = 40
NPR = N // NS

_MESH = plsc.VectorSubcoreMesh(
    core_axis_name="c", subcore_axis_name="s", num_cores=NC, num_subcores=NS)
_SC_PARAMS = pltpu.CompilerParams(needs_layout_passes=False)


def _zero_1d(ref, nwords):
    z = jnp.zeros((L,), jnp.float32)

    def body(i, _):
        ref[pl.ds(i * L, L)] = z
        return 0

    lax.fori_loop(0, nwords // L, body, 0)


@functools.partial(
    pl.kernel,
    out_type=jax.ShapeDtypeStruct((2, NPAD), jnp.float32),
    mesh=_MESH,
    compiler_params=_SC_PARAMS,
    scratch_types=[
        pltpu.VMEM_SHARED((NS, NPAD), jnp.float32),
        pltpu.VMEM((NPAD,), jnp.float32),
        pltpu.VMEM((C1,), jnp.int32),
        pltpu.VMEM((PT,), jnp.float32),
        pltpu.VMEM((PT,), jnp.float32),
    ],
)
def _norms_kernel(ei, norms, part_sh, loc, idxb, tmp, accb):
    c = lax.axis_index("c")
    s = lax.axis_index("s")
    _zero_1d(loc, NPAD)
    ones = jnp.ones((L,), jnp.float32)
    eoff = s * EPT1

    def chunk_body(q, _):
        pltpu.sync_copy(ei.at[pl.ds(c * E + eoff + q * C1, C1)], idxb)

        def inner(k, _):
            v = idxb[pl.ds(k * L, L)]
            plsc.addupdate_scatter(loc, [v], ones)
            return 0

        lax.fori_loop(0, C1 // L, inner, 0)
        return 0

    lax.fori_loop(0, EPT1 // C1, chunk_body, 0)
    pltpu.sync_copy(loc, part_sh.at[s])
    plsc.subcore_barrier()

    noff = s * PT
    pltpu.sync_copy(part_sh.at[0, pl.ds(noff, PT)], accb)

    def red(p, _):
        pltpu.sync_copy(part_sh.at[p, pl.ds(noff, PT)], tmp)

        def add_(j, _):
            sl = pl.ds(j * L, L)
            accb[sl] = accb[sl] + tmp[sl]
            return 0

        lax.fori_loop(0, PT // L, add_, 0)
        return 0

    lax.fori_loop(1, NS, red, 0)

    def nrm(j, _):
        sl = pl.ds(j * L, L)
        d = jnp.maximum(accb[sl], 1.0)
        i = plsc.bitcast(d, jnp.int32)
        i = 0x5F3759DF - lax.shift_right_logical(i, 1)
        y = plsc.bitcast(i, jnp.float32)
        for _ in range(3):
            y = y * (1.5 - 0.5 * d * y * y)
        accb[sl] = y
        return 0

    lax.fori_loop(0, PT // L, nrm, 0)
    pltpu.sync_copy(accb, norms.at[c, pl.ds(noff, PT)])


NCH = EPT3 // C3
BC = 50
NBT = NCH // BC
NG = (BC - 6) // 4


@functools.partial(
    pl.kernel,
    out_type=jax.ShapeDtypeStruct((2, NPAD, D), jnp.float32),
    mesh=_MESH,
    compiler_params=_SC_PARAMS,
    scratch_types=[
        pltpu.VMEM_SHARED((NPAD, D), jnp.float32),
        pltpu.VMEM((BC, C3), jnp.int32),
        pltpu.VMEM((BC, C3), jnp.int32),
        [pltpu.VMEM((C3, D), jnp.float32)] * 4,
        [pltpu.SemaphoreType.DMA] * 4,
        [pltpu.SemaphoreType.DMA] * 4,
    ],
)
def _agg_kernel(hs, src3, dst3, out, acc_sh, sidxb, didxb, rbufs,
                gsems, ssems):
    c = lax.axis_index("c")
    s = lax.axis_index("s")
    w = c * NS + s

    z = jnp.zeros((L,), jnp.float32)

    def zr(r, _):
        def zc(j, _):
            rbufs[0][r, pl.ds(j * L, L)] = z
            return 0

        lax.fori_loop(0, D // L, zc, 0)
        return 0

    lax.fori_loop(0, C3, zr, 0)

    def za(m, _):
        pltpu.sync_copy(rbufs[0], acc_sh.at[pl.ds(s * PT + m * C3, C3)])
        return 0

    lax.fori_loop(0, PT // C3, za, 0)
    plsc.subcore_barrier()

    def g_start(q, k):
        pltpu.async_copy(hs.at[sidxb.at[q]], rbufs[k], gsems[k])

    def g_wait(q, k):
        pltpu.make_async_copy(hs.at[sidxb.at[q]], rbufs[k], gsems[k]).wait()

    def s_start(q, k):
        pltpu.async_copy(rbufs[k], acc_sh.at[didxb.at[q]], ssems[k],
                         add=True)

    def s_wait(q, k):
        pltpu.make_async_copy(rbufs[k], acc_sh.at[didxb.at[q]],
                              ssems[k]).wait()

    def block(b, _):
        bb = w * NBT + b
        pltpu.sync_copy(src3.at[bb], sidxb)
        pltpu.sync_copy(dst3.at[bb], didxb)
        g_start(0, 0)
        g_start(1, 1)
        g_start(2, 2)
        g_wait(0, 0)
        s_start(0, 0)
        g_start(3, 3)

        def grp(g, _):
            q = 4 * g
            g_wait(q + 1, 1)
            s_start(q + 1, 1)
            s_wait(q, 0)
            g_start(q + 4, 0)
            g_wait(q + 2, 2)
            s_start(q + 2, 2)
            s_wait(q + 1, 1)
            g_start(q + 5, 1)
            g_wait(q + 3, 3)
            s_start(q + 3, 3)
            s_wait(q + 2, 2)
            g_start(q + 6, 2)
            g_wait(q + 4, 0)
            s_start(q + 4, 0)
            s_wait(q + 3, 3)
            g_start(q + 7, 3)
            return 0

        lax.fori_loop(0, NG, grp, 0)
        qe = 4 * NG
        g_wait(qe + 1, 1)
        s_start(qe + 1, 1)
        s_wait(qe, 0)
        g_start(qe + 4, 0)
        g_wait(qe + 2, 2)
        s_start(qe + 2, 2)
        s_wait(qe + 1, 1)
        g_start(qe + 5, 1)
        g_wait(qe + 3, 3)
        s_start(qe + 3, 3)
        s_wait(qe + 2, 2)
        g_wait(qe + 4, 0)
        s_start(qe + 4, 0)
        s_wait(qe + 3, 3)
        g_wait(qe + 5, 1)
        s_start(qe + 5, 1)
        s_wait(qe + 4, 0)
        s_wait(qe + 5, 1)
        return 0

    lax.fori_loop(0, NBT, block, 0)

    plsc.subcore_barrier()
    pltpu.sync_copy(acc_sh.at[pl.ds(s * PT, PT)],
                    out.at[c, pl.ds(s * PT, PT)])


_BLK = 1000


def _scale_matmul_body(x_ref, ns_ref, w_ref, o_ref):
    o_ref[...] = jnp.dot(x_ref[...] * ns_ref[...], w_ref[...],
                         preferred_element_type=jnp.float32)


def _final_body(a0_ref, a1_ref, nd_ref, b_ref, o_ref):
    o_ref[...] = (a0_ref[0] + a1_ref[0]) * nd_ref[...] + b_ref[...]


def kernel(x, edge_index, W, b):
    ei_flat = edge_index.reshape(2 * E)
    norms = _norms_kernel(ei_flat)
    ns = norms[0, :N][:, None]
    nd = norms[1, :N][:, None]

    hs = pl.pallas_call(
        _scale_matmul_body,
        grid=(N // _BLK,),
        in_specs=[
            pl.BlockSpec((_BLK, D), lambda i: (i, 0)),
            pl.BlockSpec((_BLK, 1), lambda i: (i, 0)),
            pl.BlockSpec((D, D), lambda i: (0, 0)),
        ],
        out_specs=pl.BlockSpec((_BLK, D), lambda i: (i, 0)),
        out_shape=jax.ShapeDtypeStruct((N, D), jnp.float32),
    )(x, ns, W)

    src3 = edge_index[0].reshape(NC * NS * NBT, BC, C3)
    dst3 = edge_index[1].reshape(NC * NS * NBT, BC, C3)
    agg2 = _agg_kernel(hs, src3, dst3)[:, :N, :]

    out = pl.pallas_call(
        _final_body,
        grid=(N // _BLK,),
        in_specs=[
            pl.BlockSpec((1, _BLK, D), lambda i: (0, i, 0)),
            pl.BlockSpec((1, _BLK, D), lambda i: (1, i, 0)),
            pl.BlockSpec((_BLK, 1), lambda i: (i, 0)),
            pl.BlockSpec((1, D), lambda i: (0, 0)),
        ],
        out_specs=pl.BlockSpec((_BLK, D), lambda i: (i, 0)),
        out_shape=jax.ShapeDtypeStruct((N, D), jnp.float32),
    )(agg2, agg2, nd, b.reshape(1, D))
    return out


# speedup vs baseline: 25.3701x; 1.0072x over previous
import functools

import jax
import jax.numpy as jnp
from jax import lax
from jax.experimental import pallas as pl
from jax.experimental.pallas import tpu as pltpu
from jax.experimental.pallas import tpu_sc as plsc

N = 10000
E = 320000
D = 128
DH = D // 2
NC, NS, L = 2, 16, 16
NPAD = 10240
PT = NPAD // NS
EPT1 = E // NS
C1 = 800
NC1 = EPT1 // C1
EPT3 = E // (NC * NS)
C3 = 40
NPR = N // NS

_MESH = plsc.VectorSubcoreMesh(
    core_axis_name="c", subcore_axis_name="s", num_cores=NC, num_subcores=NS)
_SC_PARAMS = pltpu.CompilerParams(needs_layout_passes=False)


def _zero_1d(ref, nwords):
    z = jnp.zeros((L,), jnp.float32)

    def body(i, _):
        ref[pl.ds(i * L, L)] = z
        return 0

    lax.fori_loop(0, nwords // L, body, 0)


@functools.partial(
    pl.kernel,
    out_type=jax.ShapeDtypeStruct((2, NPAD), jnp.float32),
    mesh=_MESH,
    compiler_params=_SC_PARAMS,
    scratch_types=[
        pltpu.VMEM_SHARED((NS, NPAD), jnp.float32),
        pltpu.VMEM((NPAD,), jnp.float32),
        pltpu.VMEM((C1,), jnp.int32),
        pltpu.VMEM((C1,), jnp.int32),
        pltpu.VMEM((PT,), jnp.float32),
        pltpu.VMEM((PT,), jnp.float32),
        pltpu.VMEM((PT,), jnp.float32),
        pltpu.SemaphoreType.DMA,
        pltpu.SemaphoreType.DMA,
    ],
)
def _norms_kernel(ei, norms, part_sh, loc, idxb0, idxb1, tmp0, tmp1, accb,
                  dsem0, dsem1):
    c = lax.axis_index("c")
    s = lax.axis_index("s")
    _zero_1d(loc, NPAD)
    ones = jnp.ones((L,), jnp.float32)
    eoff = s * EPT1

    def i_start(q, buf, sem):
        pltpu.async_copy(ei.at[pl.ds(c * E + eoff + q * C1, C1)], buf, sem)

    def i_wait(q, buf, sem):
        pltpu.make_async_copy(
            ei.at[pl.ds(c * E + eoff + q * C1, C1)], buf, sem).wait()

    def hist(buf):
        def inner(j, _):
            base = j * 5 * L
            for k in range(5):
                v = buf[pl.ds(base + k * L, L)]
                plsc.addupdate_scatter(loc, [v], ones)
            return 0

        lax.fori_loop(0, C1 // (5 * L), inner, 0)

    i_start(0, idxb0, dsem0)

    def chunk_pair(i, _):
        i_wait(2 * i, idxb0, dsem0)
        i_start(2 * i + 1, idxb1, dsem1)
        hist(idxb0)
        i_wait(2 * i + 1, idxb1, dsem1)
        i_start(2 * i + 2, idxb0, dsem0)
        hist(idxb1)
        return 0

    lax.fori_loop(0, (NC1 - 1) // 2, chunk_pair, 0)
    i_wait(NC1 - 1, idxb0, dsem0)
    hist(idxb0)
    pltpu.sync_copy(loc, part_sh.at[s])
    plsc.subcore_barrier()

    noff = s * PT
    pltpu.sync_copy(part_sh.at[0, pl.ds(noff, PT)], accb)

    def p_start(p, buf, sem):
        pltpu.async_copy(part_sh.at[p, pl.ds(noff, PT)], buf, sem)

    def p_wait(p, buf, sem):
        pltpu.make_async_copy(
            part_sh.at[p, pl.ds(noff, PT)], buf, sem).wait()

    def acc_add(buf):
        def add_(j, _):
            base = j * 4 * L
            for k in range(4):
                sl = pl.ds(base + k * L, L)
                accb[sl] = accb[sl] + buf[sl]
            return 0

        lax.fori_loop(0, PT // (4 * L), add_, 0)

    p_start(1, tmp0, dsem0)

    def red_pair(i, _):
        p = 2 * i + 1
        p_wait(p, tmp0, dsem0)
        p_start(p + 1, tmp1, dsem1)
        acc_add(tmp0)
        p_wait(p + 1, tmp1, dsem1)
        p_start(p + 2, tmp0, dsem0)
        acc_add(tmp1)
        return 0

    lax.fori_loop(0, (NS - 2) // 2, red_pair, 0)
    p_wait(NS - 1, tmp0, dsem0)
    acc_add(tmp0)

    def nrm(j, _):
        sl = pl.ds(j * L, L)
        d = jnp.maximum(accb[sl], 1.0)
        i = plsc.bitcast(d, jnp.int32)
        i = 0x5F3759DF - lax.shift_right_logical(i, 1)
        y = plsc.bitcast(i, jnp.float32)
        for _ in range(3):
            y = y * (1.5 - 0.5 * d * y * y)
        accb[sl] = y
        return 0

    lax.fori_loop(0, PT // L, nrm, 0)
    pltpu.sync_copy(accb, norms.at[c, pl.ds(noff, PT)])


NCH = EPT3 // C3
BC = 50
NBT = NCH // BC
NG = (BC - 6) // 4


@functools.partial(
    pl.kernel,
    out_type=jax.ShapeDtypeStruct((2, NPAD, D), jnp.float32),
    mesh=_MESH,
    compiler_params=_SC_PARAMS,
    scratch_types=[
        pltpu.VMEM_SHARED((NPAD, D), jnp.float32),
        pltpu.VMEM((BC, C3), jnp.int32),
        pltpu.VMEM((BC, C3), jnp.int32),
        [pltpu.VMEM((C3, D), jnp.float32)] * 4,
        [pltpu.SemaphoreType.DMA] * 4,
        [pltpu.SemaphoreType.DMA] * 4,
    ],
)
def _agg_kernel(hs, src3, dst3, out, acc_sh, sidxb, didxb, rbufs,
                gsems, ssems):
    c = lax.axis_index("c")
    s = lax.axis_index("s")
    w = c * NS + s

    z = jnp.zeros((L,), jnp.float32)

    def zr(r, _):
        def zc(j, _):
            rbufs[0][r, pl.ds(j * L, L)] = z
            return 0

        lax.fori_loop(0, D // L, zc, 0)
        return 0

    lax.fori_loop(0, C3, zr, 0)

    def za(m, _):
        pltpu.sync_copy(rbufs[0], acc_sh.at[pl.ds(s * PT + m * C3, C3)])
        return 0

    lax.fori_loop(0, PT // C3, za, 0)
    plsc.subcore_barrier()

    def g_start(q, k):
        pltpu.async_copy(hs.at[sidxb.at[q]], rbufs[k], gsems[k])

    def g_wait(q, k):
        pltpu.make_async_copy(hs.at[sidxb.at[q]], rbufs[k], gsems[k]).wait()

    def s_start(q, k):
        pltpu.async_copy(rbufs[k], acc_sh.at[didxb.at[q]], ssems[k],
                         add=True)

    def s_wait(q, k):
        pltpu.make_async_copy(rbufs[k], acc_sh.at[didxb.at[q]],
                              ssems[k]).wait()

    def block(b, _):
        bb = w * NBT + b
        pltpu.sync_copy(src3.at[bb], sidxb)
        pltpu.sync_copy(dst3.at[bb], didxb)
        g_start(0, 0)
        g_start(1, 1)
        g_start(2, 2)
        g_wait(0, 0)
        s_start(0, 0)
        g_start(3, 3)

        def grp(g, _):
            q = 4 * g
            g_wait(q + 1, 1)
            s_start(q + 1, 1)
            s_wait(q, 0)
            g_start(q + 4, 0)
            g_wait(q + 2, 2)
            s_start(q + 2, 2)
            s_wait(q + 1, 1)
            g_start(q + 5, 1)
            g_wait(q + 3, 3)
            s_start(q + 3, 3)
            s_wait(q + 2, 2)
            g_start(q + 6, 2)
            g_wait(q + 4, 0)
            s_start(q + 4, 0)
            s_wait(q + 3, 3)
            g_start(q + 7, 3)
            return 0

        lax.fori_loop(0, NG, grp, 0)
        qe = 4 * NG
        g_wait(qe + 1, 1)
        s_start(qe + 1, 1)
        s_wait(qe, 0)
        g_start(qe + 4, 0)
        g_wait(qe + 2, 2)
        s_start(qe + 2, 2)
        s_wait(qe + 1, 1)
        g_start(qe + 5, 1)
        g_wait(qe + 3, 3)
        s_start(qe + 3, 3)
        s_wait(qe + 2, 2)
        g_wait(qe + 4, 0)
        s_start(qe + 4, 0)
        s_wait(qe + 3, 3)
        g_wait(qe + 5, 1)
        s_start(qe + 5, 1)
        s_wait(qe + 4, 0)
        s_wait(qe + 5, 1)
        return 0

    lax.fori_loop(0, NBT, block, 0)

    plsc.subcore_barrier()
    pltpu.sync_copy(acc_sh.at[pl.ds(s * PT, PT)],
                    out.at[c, pl.ds(s * PT, PT)])


_BLK = 1000


def _scale_matmul_body(x_ref, ns_ref, w_ref, o_ref):
    o_ref[...] = jnp.dot(x_ref[...] * ns_ref[...], w_ref[...],
                         preferred_element_type=jnp.float32)


def _final_body(a0_ref, a1_ref, nd_ref, b_ref, o_ref):
    o_ref[...] = (a0_ref[0] + a1_ref[0]) * nd_ref[...] + b_ref[...]


def kernel(x, edge_index, W, b):
    ei_flat = edge_index.reshape(2 * E)
    norms = _norms_kernel(ei_flat)
    ns = norms[0, :N][:, None]
    nd = norms[1, :N][:, None]

    hs = pl.pallas_call(
        _scale_matmul_body,
        grid=(N // _BLK,),
        in_specs=[
            pl.BlockSpec((_BLK, D), lambda i: (i, 0)),
            pl.BlockSpec((_BLK, 1), lambda i: (i, 0)),
            pl.BlockSpec((D, D), lambda i: (0, 0)),
        ],
        out_specs=pl.BlockSpec((_BLK, D), lambda i: (i, 0)),
        out_shape=jax.ShapeDtypeStruct((N, D), jnp.float32),
    )(x, ns, W)

    src3 = edge_index[0].reshape(NC * NS * NBT, BC, C3)
    dst3 = edge_index[1].reshape(NC * NS * NBT, BC, C3)
    agg2 = _agg_kernel(hs, src3, dst3)[:, :N, :]

    out = pl.pallas_call(
        _final_body,
        grid=(N // _BLK,),
        in_specs=[
            pl.BlockSpec((1, _BLK, D), lambda i: (0, i, 0)),
            pl.BlockSpec((1, _BLK, D), lambda i: (1, i, 0)),
            pl.BlockSpec((_BLK, 1), lambda i: (i, 0)),
            pl.BlockSpec((1, D), lambda i: (0, 0)),
        ],
        out_specs=pl.BlockSpec((_BLK, D), lambda i: (i, 0)),
        out_shape=jax.ShapeDtypeStruct((N, D), jnp.float32),
    )(agg2, agg2, nd, b.reshape(1, D))
    return out


# speedup vs baseline: 27.2306x; 1.0733x over previous
import functools

import jax
import jax.numpy as jnp
from jax import lax
from jax.experimental import pallas as pl
from jax.experimental.pallas import tpu as pltpu
from jax.experimental.pallas import tpu_sc as plsc

N = 10000
E = 320000
D = 128
DH = D // 2
NC, NS, L = 2, 16, 16
NPAD = 10240
PT = NPAD // NS
EPT1 = E // NS
C1 = 800
NC1 = EPT1 // C1
EPT3 = E // (NC * NS)
C3 = 40
NPR = N // NS

_MESH = plsc.VectorSubcoreMesh(
    core_axis_name="c", subcore_axis_name="s", num_cores=NC, num_subcores=NS)
_SC_PARAMS = pltpu.CompilerParams(needs_layout_passes=False)


def _zero_1d(ref, nwords):
    z = jnp.zeros((L,), jnp.float32)

    def body(i, _):
        ref[pl.ds(i * L, L)] = z
        return 0

    lax.fori_loop(0, nwords // L, body, 0)


@functools.partial(
    pl.kernel,
    out_type=jax.ShapeDtypeStruct((2, NPAD), jnp.float32),
    mesh=_MESH,
    compiler_params=_SC_PARAMS,
    scratch_types=[
        pltpu.VMEM_SHARED((NS, NPAD), jnp.float32),
        pltpu.VMEM((NPAD,), jnp.float32),
        pltpu.VMEM((C1,), jnp.int32),
        pltpu.VMEM((C1,), jnp.int32),
        pltpu.VMEM((PT,), jnp.float32),
        pltpu.VMEM((PT,), jnp.float32),
        pltpu.VMEM((PT,), jnp.float32),
        pltpu.SemaphoreType.DMA,
        pltpu.SemaphoreType.DMA,
    ],
)
def _norms_kernel(ei, norms, part_sh, loc, idxb0, idxb1, tmp0, tmp1, accb,
                  dsem0, dsem1):
    c = lax.axis_index("c")
    s = lax.axis_index("s")
    _zero_1d(loc, NPAD)
    ones = jnp.ones((L,), jnp.float32)
    eoff = s * EPT1

    def i_start(q, buf, sem):
        pltpu.async_copy(ei.at[pl.ds(c * E + eoff + q * C1, C1)], buf, sem)

    def i_wait(q, buf, sem):
        pltpu.make_async_copy(
            ei.at[pl.ds(c * E + eoff + q * C1, C1)], buf, sem).wait()

    def hist(buf):
        def inner(j, _):
            base = j * 5 * L
            for k in range(5):
                v = buf[pl.ds(base + k * L, L)]
                plsc.addupdate_scatter(loc, [v], ones)
            return 0

        lax.fori_loop(0, C1 // (5 * L), inner, 0)

    i_start(0, idxb0, dsem0)

    def chunk_pair(i, _):
        i_wait(2 * i, idxb0, dsem0)
        i_start(2 * i + 1, idxb1, dsem1)
        hist(idxb0)
        i_wait(2 * i + 1, idxb1, dsem1)
        i_start(2 * i + 2, idxb0, dsem0)
        hist(idxb1)
        return 0

    lax.fori_loop(0, (NC1 - 1) // 2, chunk_pair, 0)
    i_wait(NC1 - 1, idxb0, dsem0)
    hist(idxb0)
    pltpu.sync_copy(loc, part_sh.at[s])
    plsc.subcore_barrier()

    noff = s * PT
    pltpu.sync_copy(part_sh.at[0, pl.ds(noff, PT)], accb)

    def p_start(p, buf, sem):
        pltpu.async_copy(part_sh.at[p, pl.ds(noff, PT)], buf, sem)

    def p_wait(p, buf, sem):
        pltpu.make_async_copy(
            part_sh.at[p, pl.ds(noff, PT)], buf, sem).wait()

    def acc_add(buf):
        def add_(j, _):
            base = j * 4 * L
            for k in range(4):
                sl = pl.ds(base + k * L, L)
                accb[sl] = accb[sl] + buf[sl]
            return 0

        lax.fori_loop(0, PT // (4 * L), add_, 0)

    p_start(1, tmp0, dsem0)

    def red_pair(i, _):
        p = 2 * i + 1
        p_wait(p, tmp0, dsem0)
        p_start(p + 1, tmp1, dsem1)
        acc_add(tmp0)
        p_wait(p + 1, tmp1, dsem1)
        p_start(p + 2, tmp0, dsem0)
        acc_add(tmp1)
        return 0

    lax.fori_loop(0, (NS - 2) // 2, red_pair, 0)
    p_wait(NS - 1, tmp0, dsem0)
    acc_add(tmp0)

    def nrm(j, _):
        sl = pl.ds(j * L, L)
        d = jnp.maximum(accb[sl], 1.0)
        i = plsc.bitcast(d, jnp.int32)
        i = 0x5F3759DF - lax.shift_right_logical(i, 1)
        y = plsc.bitcast(i, jnp.float32)
        for _ in range(3):
            y = y * (1.5 - 0.5 * d * y * y)
        accb[sl] = y
        return 0

    lax.fori_loop(0, PT // L, nrm, 0)
    pltpu.sync_copy(accb, norms.at[c, pl.ds(noff, PT)])


NCH = EPT3 // C3
BC = 50
NBT = NCH // BC
NG = (BC - 6) // 4


@functools.partial(
    pl.kernel,
    out_type=jax.ShapeDtypeStruct((2, NPAD, D), jnp.float32),
    mesh=_MESH,
    compiler_params=_SC_PARAMS,
    scratch_types=[
        pltpu.VMEM_SHARED((NPAD, D), jnp.float32),
        pltpu.VMEM((BC, C3), jnp.int32),
        pltpu.VMEM((BC, C3), jnp.int32),
        [pltpu.VMEM((C3, D), jnp.float32)] * 4,
        [pltpu.SemaphoreType.DMA] * 4,
        [pltpu.SemaphoreType.DMA] * 4,
    ],
)
def _agg_kernel(hs, ei3, out, acc_sh, sidxb, didxb, rbufs,
                gsems, ssems):
    c = lax.axis_index("c")
    s = lax.axis_index("s")
    w = c * NS + s

    z = jnp.zeros((L,), jnp.float32)

    def zr(r, _):
        def zc(j, _):
            rbufs[0][r, pl.ds(j * L, L)] = z
            return 0

        lax.fori_loop(0, D // L, zc, 0)
        return 0

    lax.fori_loop(0, C3, zr, 0)

    def za(m, _):
        pltpu.sync_copy(rbufs[0], acc_sh.at[pl.ds(s * PT + m * C3, C3)])
        return 0

    lax.fori_loop(0, PT // C3, za, 0)
    plsc.subcore_barrier()

    def g_start(q, k):
        pltpu.async_copy(hs.at[sidxb.at[q]], rbufs[k], gsems[k])

    def g_wait(q, k):
        pltpu.make_async_copy(hs.at[sidxb.at[q]], rbufs[k], gsems[k]).wait()

    def s_start(q, k):
        pltpu.async_copy(rbufs[k], acc_sh.at[didxb.at[q]], ssems[k],
                         add=True)

    def s_wait(q, k):
        pltpu.make_async_copy(rbufs[k], acc_sh.at[didxb.at[q]],
                              ssems[k]).wait()

    def block(b, _):
        bb = w * NBT + b
        pltpu.sync_copy(ei3.at[bb], sidxb)
        pltpu.sync_copy(ei3.at[NC * NS * NBT + bb], didxb)
        g_start(0, 0)
        g_start(1, 1)
        g_start(2, 2)
        g_wait(0, 0)
        s_start(0, 0)
        g_start(3, 3)

        def grp(g, _):
            q = 4 * g
            g_wait(q + 1, 1)
            s_start(q + 1, 1)
            s_wait(q, 0)
            g_start(q + 4, 0)
            g_wait(q + 2, 2)
            s_start(q + 2, 2)
            s_wait(q + 1, 1)
            g_start(q + 5, 1)
            g_wait(q + 3, 3)
            s_start(q + 3, 3)
            s_wait(q + 2, 2)
            g_start(q + 6, 2)
            g_wait(q + 4, 0)
            s_start(q + 4, 0)
            s_wait(q + 3, 3)
            g_start(q + 7, 3)
            return 0

        lax.fori_loop(0, NG, grp, 0)
        qe = 4 * NG
        g_wait(qe + 1, 1)
        s_start(qe + 1, 1)
        s_wait(qe, 0)
        g_start(qe + 4, 0)
        g_wait(qe + 2, 2)
        s_start(qe + 2, 2)
        s_wait(qe + 1, 1)
        g_start(qe + 5, 1)
        g_wait(qe + 3, 3)
        s_start(qe + 3, 3)
        s_wait(qe + 2, 2)
        g_wait(qe + 4, 0)
        s_start(qe + 4, 0)
        s_wait(qe + 3, 3)
        g_wait(qe + 5, 1)
        s_start(qe + 5, 1)
        s_wait(qe + 4, 0)
        s_wait(qe + 5, 1)
        return 0

    lax.fori_loop(0, NBT, block, 0)

    plsc.subcore_barrier()
    pltpu.sync_copy(acc_sh.at[pl.ds(s * PT, PT)],
                    out.at[c, pl.ds(s * PT, PT)])


_BLK = 1024


def _scale_matmul_body(x_ref, nrm_ref, w_ref, o_ref):
    o_ref[...] = jnp.dot(x_ref[...] * nrm_ref[0][:, None], w_ref[...],
                         preferred_element_type=jnp.float32)


def _final_body(a0_ref, a1_ref, nrm_ref, b_ref, o_ref):
    o_ref[...] = (a0_ref[0] + a1_ref[0]) * nrm_ref[1][:, None] + b_ref[...]


def kernel(x, edge_index, W, b):
    norms = _norms_kernel(edge_index.reshape(2 * E))

    hs = pl.pallas_call(
        _scale_matmul_body,
        grid=(NPAD // _BLK,),
        in_specs=[
            pl.BlockSpec((_BLK, D), lambda i: (i, 0)),
            pl.BlockSpec((2, _BLK), lambda i: (0, i)),
            pl.BlockSpec((D, D), lambda i: (0, 0)),
        ],
        out_specs=pl.BlockSpec((_BLK, D), lambda i: (i, 0)),
        out_shape=jax.ShapeDtypeStruct((N, D), jnp.float32),
    )(x, norms, W)

    ei3 = edge_index.reshape(2 * NC * NS * NBT, BC, C3)
    agg2 = _agg_kernel(hs, ei3)

    out = pl.pallas_call(
        _final_body,
        grid=(NPAD // _BLK,),
        in_specs=[
            pl.BlockSpec((1, _BLK, D), lambda i: (0, i, 0)),
            pl.BlockSpec((1, _BLK, D), lambda i: (1, i, 0)),
            pl.BlockSpec((2, _BLK), lambda i: (0, i)),
            pl.BlockSpec((1, D), lambda i: (0, 0)),
        ],
        out_specs=pl.BlockSpec((_BLK, D), lambda i: (i, 0)),
        out_shape=jax.ShapeDtypeStruct((N, D), jnp.float32),
    )(agg2, agg2, norms, b.reshape(1, D))
    return out
